# baseline (speedup 1.0000x reference)
import jax
import jax.numpy as jnp
from jax.experimental import pallas as pl
from jax.experimental.pallas import tpu as pltpu

N = 1024
IN = 128
OUT = 128
B = 256
NB = N // B
NEG_LOG2E = -1.4426950408889634


def _dot(a, b, dims):
    return jax.lax.dot_general(a, b, (dims, ((), ())),
                               preferred_element_type=jnp.float32)


def _body(x_ref, aa_hbm, ab_hbm, wg_ref, bg_ref, wn_ref, an_ref,
          wa_ref, ba_ref, wb_ref, bb_ref, out_hbm,
          aa_s, ab_s, dense_s, out_s, ga_s, xa_s, sem_a, sem_b, sem_o):
    bf = jnp.bfloat16

    cps_a = [pltpu.make_async_copy(aa_hbm.at[pl.ds(k * B, B), :],
                                   aa_s.at[pl.ds(k * B, B), :],
                                   sem_a.at[k]) for k in range(NB)]
    cps_b = [pltpu.make_async_copy(ab_hbm.at[pl.ds(k * B, B), :],
                                   ab_s.at[pl.ds(k * B, B), :],
                                   sem_b.at[k]) for k in range(NB)]
    cps_o = [pltpu.make_async_copy(out_s.at[pl.ds(k * B, B), :],
                                   out_hbm.at[pl.ds(k * B, B), :],
                                   sem_o.at[k]) for k in range(NB)]
    for k in range(NB):
        cps_a[k].start()

    x = x_ref[...]
    xh = _dot(x, wn_ref[...], ((1,), (0,)))
    xg = _dot(x, wg_ref[...], ((1,), (0,)))
    an = an_ref[...]
    ps = _dot(xh, an[:, :OUT], ((1,), (1,))) * NEG_LOG2E
    pd = _dot(an[:, OUT:], xh, ((1,), (1,))) * NEG_LOG2E
    va = _dot(wa_ref[:, :IN], x, ((1,), (1,)))
    vb = _dot(wb_ref[:, :IN], x, ((1,), (1,)))
    u_a = _dot(x, wa_ref[:, IN:], ((1,), (1,)))
    u_b = _dot(x, wb_ref[:, IN:], ((1,), (1,)))
    xgx = jnp.concatenate(
        [xg, jnp.broadcast_to(vb.reshape(N, 1), (N, OUT))], axis=1).astype(bf)

    r_parts = []
    for k in range(NB):
        rows = pl.ds(k * B, B)
        sl = slice(k * B, (k + 1) * B)
        cps_a[k].wait()
        cps_b[k].start()
        aa = aa_s[rows, :]
        pm = ps[sl, :] + pd
        e = jnp.exp2(jnp.minimum(pm, 0.01 * pm))
        d = aa * e
        dense_s[rows, :] = d.astype(bf)
        r_parts.append(jnp.sum(d, axis=1, keepdims=True))
        m_a = jnp.sum(aa * va, axis=1, keepdims=True)
        ga_s[rows, :] = jax.nn.sigmoid(m_a + u_a[sl, :] + ba_ref[0])

    r = jnp.concatenate(r_parts, axis=0)
    m1 = (xh * (1.0 / (r + 1e-05))).astype(bf)
    for k in range(NB):
        rows = pl.ds(k * B, B)
        xa_s[rows, :] = _dot(dense_s[rows, :], m1, ((1,), (0,)))

    for k in range(NB):
        rows = pl.ds(k * B, B)
        sl = slice(k * B, (k + 1) * B)
        cps_b[k].wait()
        ab = ab_s[rows, :]
        y = _dot(ab.astype(bf), xgx, ((1,), (0,)))
        xbb = y[:, :OUT] + bg_ref[...]
        m_b = y[:, OUT:OUT + 1]
        gate_b = jax.nn.sigmoid(m_b + u_b[sl, :] + bb_ref[0])
        out_s[rows, :] = jax.nn.sigmoid(ga_s[rows, :] * xa_s[rows, :]
                                        + gate_b * xbb)
        cps_o[k].start()

    for k in range(NB):
        cps_o[k].wait()


@jax.jit
def kernel(x, adj_a, adj_b, W_gcn, b_gcn, W_na, a_na, Wa, ba, Wb, bb):
    f32 = jnp.float32
    bf = jnp.bfloat16

    vmem = lambda: pl.BlockSpec(memory_space=pltpu.MemorySpace.VMEM)
    return pl.pallas_call(
        _body,
        in_specs=[
            vmem(),
            pl.BlockSpec(memory_space=pltpu.MemorySpace.HBM),
            pl.BlockSpec(memory_space=pltpu.MemorySpace.HBM),
            vmem(),
            vmem(),
            vmem(),
            vmem(),
            vmem(),
            pl.BlockSpec(memory_space=pltpu.MemorySpace.SMEM),
            vmem(),
            pl.BlockSpec(memory_space=pltpu.MemorySpace.SMEM),
        ],
        out_specs=pl.BlockSpec(memory_space=pltpu.MemorySpace.HBM),
        out_shape=jax.ShapeDtypeStruct((N, OUT), f32),
        scratch_shapes=[
            pltpu.VMEM((N, N), f32),
            pltpu.VMEM((N, N), f32),
            pltpu.VMEM((N, N), bf),
            pltpu.VMEM((N, OUT), f32),
            pltpu.VMEM((N, 1), f32),
            pltpu.VMEM((N, OUT), f32),
            pltpu.SemaphoreType.DMA((NB,)),
            pltpu.SemaphoreType.DMA((NB,)),
            pltpu.SemaphoreType.DMA((NB,)),
        ],
    )(x, adj_a, adj_b, W_gcn, b_gcn.reshape(1, OUT), W_na, a_na,
      Wa, ba, Wb, bb)


# speedup vs baseline: 1.1011x; 1.1011x over previous
import jax
import jax.numpy as jnp
from jax.experimental import pallas as pl
from jax.experimental.pallas import tpu as pltpu

N = 1024
IN = 128
OUT = 128
B = 256
NB = N // B
NEG_LOG2E = -1.4426950408889634


def _dot(a, b, dims):
    return jax.lax.dot_general(a, b, (dims, ((), ())),
                               preferred_element_type=jnp.float32)


def _body(x_ref, aa_hbm, ab_hbm, wg_ref, bg_ref, wn_ref, an_ref,
          wa_ref, ba_ref, wb_ref, bb_ref, out_ref,
          aa_s, ab_s, dense_s, xbb_s, ga_s, sem_a, sem_b):
    bf = jnp.bfloat16

    cps_a = [pltpu.make_async_copy(aa_hbm.at[pl.ds(k * B, B), :],
                                   aa_s.at[pl.ds(k * B, B), :],
                                   sem_a.at[k]) for k in range(NB)]
    cps_b = [pltpu.make_async_copy(ab_hbm.at[pl.ds(k * B, B), :],
                                   ab_s.at[pl.ds(k * B, B), :],
                                   sem_b.at[k]) for k in range(NB)]
    for k in range(NB):
        cps_a[k].start()
        cps_b[k].start()

    x = x_ref[...]
    xh = _dot(x, wn_ref[...], ((1,), (0,)))
    xg = _dot(x, wg_ref[...], ((1,), (0,))).astype(bf)
    an = an_ref[...]
    ps = _dot(xh, an[:, :OUT], ((1,), (1,))) * NEG_LOG2E
    pd = _dot(an[:, OUT:], xh, ((1,), (1,))) * NEG_LOG2E
    va = _dot(wa_ref[:, :IN], x, ((1,), (1,)))
    vb = _dot(wb_ref[:, :IN], x, ((1,), (1,)))
    u_a = _dot(x, wa_ref[:, IN:], ((1,), (1,)))
    u_b = _dot(x, wb_ref[:, IN:], ((1,), (1,)))

    r_parts = []
    for k in range(NB):
        rows = pl.ds(k * B, B)
        sl = slice(k * B, (k + 1) * B)

        cps_a[k].wait()
        aa = aa_s[rows, :]
        pm = ps[sl, :] + pd
        e = jnp.exp2(jnp.minimum(pm, 0.01 * pm))
        d = aa * e
        dense_s[rows, :] = d.astype(bf)
        r_parts.append(jnp.sum(d, axis=1, keepdims=True))
        m_a = jnp.sum(aa * va, axis=1, keepdims=True)
        ga_s[rows, :] = jax.nn.sigmoid(m_a + u_a[sl, :] + ba_ref[0])

        cps_b[k].wait()
        ab = ab_s[rows, :]
        xbb = _dot(ab.astype(bf), xg, ((1,), (0,))) + bg_ref[...]
        m_b = jnp.sum(ab * vb, axis=1, keepdims=True)
        gate_b = jax.nn.sigmoid(m_b + u_b[sl, :] + bb_ref[0])
        xbb_s[rows, :] = gate_b * xbb

    r = jnp.concatenate(r_parts, axis=0)
    m1 = (xh * (1.0 / (r + 1e-05))).astype(bf)

    for k in range(NB):
        rows = pl.ds(k * B, B)
        x_a = _dot(dense_s[rows, :], m1, ((1,), (0,)))
        out_ref[rows, :] = jax.nn.sigmoid(ga_s[rows, :] * x_a + xbb_s[rows, :])


@jax.jit
def kernel(x, adj_a, adj_b, W_gcn, b_gcn, W_na, a_na, Wa, ba, Wb, bb):
    f32 = jnp.float32
    bf = jnp.bfloat16

    vmem = lambda: pl.BlockSpec(memory_space=pltpu.MemorySpace.VMEM)
    return pl.pallas_call(
        _body,
        in_specs=[
            vmem(),
            pl.BlockSpec(memory_space=pltpu.MemorySpace.HBM),
            pl.BlockSpec(memory_space=pltpu.MemorySpace.HBM),
            vmem(),
            vmem(),
            vmem(),
            vmem(),
            vmem(),
            pl.BlockSpec(memory_space=pltpu.MemorySpace.SMEM),
            vmem(),
            pl.BlockSpec(memory_space=pltpu.MemorySpace.SMEM),
        ],
        out_specs=vmem(),
        out_shape=jax.ShapeDtypeStruct((N, OUT), f32),
        scratch_shapes=[
            pltpu.VMEM((N, N), f32),
            pltpu.VMEM((N, N), f32),
            pltpu.VMEM((N, N), bf),
            pltpu.VMEM((N, OUT), f32),
            pltpu.VMEM((N, 1), f32),
            pltpu.SemaphoreType.DMA((NB,)),
            pltpu.SemaphoreType.DMA((NB,)),
        ],
    )(x, adj_a, adj_b, W_gcn, b_gcn.reshape(1, OUT), W_na, a_na,
      Wa, ba, Wb, bb)
